# baseline (speedup 1.0000x reference)
import functools

import jax
import jax.numpy as jnp
from jax import lax
from jax.experimental import pallas as pl
from jax.experimental.pallas import tpu as pltpu
from jax.experimental.pallas import tpu_sc as plsc

SHARED_SCALE = 0.1


def _router_body(E, CAP, CAP2, C, x_ref, wr_ref, ws1_ref, bs1_ref, ws2_ref,
                 bs2_ref, dd_ref, dg_ref, sc16_ref, sh_ref, counts_ref,
                 tril_ref):
    step = pl.program_id(0)

    @pl.when(step == 0)
    def _init():
        counts_ref[...] = jnp.zeros_like(counts_ref)
        r = lax.broadcasted_iota(jnp.int32, (C, C), 0)
        c = lax.broadcasted_iota(jnp.int32, (C, C), 1)
        tril_ref[...] = (r >= c).astype(jnp.float32)

    x = x_ref[...]
    logits = jnp.dot(x, wr_ref[...],
                     preferred_element_type=jnp.float32)
    m = jnp.max(logits, axis=1, keepdims=True)
    iota_e = lax.broadcasted_iota(jnp.int32, logits.shape, 1)
    eidx = jnp.min(jnp.where(logits == m, iota_e, E), axis=1,
                   keepdims=True)
    gate = 1.0 / jnp.sum(jnp.exp(logits - m), axis=1, keepdims=True)

    oh = (iota_e == eidx).astype(jnp.float32)
    cum = jnp.dot(tril_ref[...], oh,
                  preferred_element_type=jnp.float32)
    pos = jnp.sum(cum * oh, axis=1, keepdims=True)
    base = jnp.sum(counts_ref[...] * oh, axis=1, keepdims=True)
    counts_ref[...] += jnp.sum(oh, axis=0, keepdims=True)

    sloti = (base + pos - 1.0).astype(jnp.int32)
    keep = sloti < CAP
    slot_c = jnp.minimum(sloti, CAP - 1)
    dd_ref[...] = eidx * CAP2 + jnp.minimum(sloti, CAP)
    dg_ref[...] = eidx * CAP + slot_c
    sc16_ref[...] = jnp.broadcast_to(jnp.where(keep, gate, 0.0),
                                     (C, 16))
    h = jnp.dot(x, ws1_ref[...], preferred_element_type=jnp.float32) + bs1_ref[0]
    h = h * jax.nn.sigmoid(h)
    sh_ref[...] = SHARED_SCALE * (
        jnp.dot(h, ws2_ref[...], preferred_element_type=jnp.float32)
        + bs2_ref[0])


def _run_router(x, Wr, Ws1, bs1, Ws2, bs2, E, CAP, CAP2, C):
    T, D = x.shape
    F = Ws1.shape[-1]
    body = functools.partial(_router_body, E, CAP, CAP2, C)
    out_shape = [
        jax.ShapeDtypeStruct((T, 1), jnp.int32),
        jax.ShapeDtypeStruct((T, 1), jnp.int32),
        jax.ShapeDtypeStruct((T, 16), jnp.float32),
        jax.ShapeDtypeStruct((T, D), jnp.float32),
    ]
    col_spec = pl.BlockSpec((C, 1), lambda i: (i, 0))
    dd, dg, sc16, sh = pl.pallas_call(
        body,
        grid=(T // C,),
        in_specs=[
            pl.BlockSpec((C, D), lambda i: (i, 0)),
            pl.BlockSpec((D, E), lambda i: (0, 0)),
            pl.BlockSpec((D, F), lambda i: (0, 0)),
            pl.BlockSpec((1, 1, F), lambda i: (0, 0, 0)),
            pl.BlockSpec((F, D), lambda i: (0, 0)),
            pl.BlockSpec((1, 1, D), lambda i: (0, 0, 0)),
        ],
        out_specs=[col_spec, col_spec,
                   pl.BlockSpec((C, 16), lambda i: (i, 0)),
                   pl.BlockSpec((C, D), lambda i: (i, 0))],
        out_shape=out_shape,
        scratch_shapes=[pltpu.VMEM((1, E), jnp.float32),
                        pltpu.VMEM((C, C), jnp.float32)],
    )(x, Wr, Ws1, bs1.reshape(1, 1, F), Ws2, bs2.reshape(1, 1, D))
    return dd.reshape(T), dg.reshape(T), sc16, sh


def _sc_scatter_rows(x, idx, n_rows_out):
    T, D = x.shape
    info = plsc.get_sparse_core_info()
    NW = info.num_cores * info.num_subcores
    ROWS = 32
    per_w = T // NW
    nchunks = per_w // ROWS
    mesh = plsc.VectorSubcoreMesh(core_axis_name="c", subcore_axis_name="s")

    @functools.partial(
        pl.kernel,
        mesh=mesh,
        out_type=jax.ShapeDtypeStruct((n_rows_out, D), x.dtype),
        scratch_types=[
            pltpu.VMEM((ROWS,), jnp.int32),
            pltpu.VMEM((ROWS,), jnp.int32),
            pltpu.VMEM((ROWS, D), x.dtype),
            pltpu.VMEM((ROWS, D), x.dtype),
            pltpu.SemaphoreType.DMA,
            pltpu.SemaphoreType.DMA,
            pltpu.SemaphoreType.DMA,
            pltpu.SemaphoreType.DMA,
        ],
    )
    def k(x_hbm, idx_hbm, out_hbm, idx_v0, idx_v1, rows_v0, rows_v1,
          lsem0, lsem1, ssem0, ssem1):
        wid = lax.axis_index("s") * info.num_cores + lax.axis_index("c")
        w_base = wid * per_w
        idx_v = (idx_v0, idx_v1)
        rows_v = (rows_v0, rows_v1)
        lsem = (lsem0, lsem1)
        ssem = (ssem0, ssem1)

        def load(i):
            base = pl.multiple_of(w_base + i * ROWS, ROWS)
            b = i % 2
            hi = pltpu.async_copy(idx_hbm.at[pl.ds(base, ROWS)],
                                  idx_v[b], lsem[b])
            hr = pltpu.async_copy(x_hbm.at[pl.ds(base, ROWS)], rows_v[b],
                                  lsem[b])
            return hi, hr

        handles = load(0)
        scat = [None, None]
        for i in range(nchunks):
            b = i % 2
            handles[0].wait()
            handles[1].wait()
            scat[b] = pltpu.async_copy(rows_v[b], out_hbm.at[idx_v[b]],
                                       ssem[b])
            if i + 1 < nchunks:
                if scat[1 - b] is not None:
                    scat[1 - b].wait()
                handles = load(i + 1)
        if nchunks > 1:
            scat[(nchunks - 2) % 2].wait()
        scat[(nchunks - 1) % 2].wait()

    return k(x, idx)


def _sc_gather_combine(table, idx, sc16, shared):
    _, D = table.shape
    T = idx.shape[0]
    info = plsc.get_sparse_core_info()
    NW = info.num_cores * info.num_subcores
    ROWS = 16
    per_w = T // NW
    npairs = per_w // (2 * ROWS)
    mesh = plsc.VectorSubcoreMesh(core_axis_name="c", subcore_axis_name="s")

    @functools.partial(
        pl.kernel,
        mesh=mesh,
        out_type=jax.ShapeDtypeStruct((T, D), jnp.float32),
        scratch_types=[
            pltpu.VMEM((per_w,), jnp.int32),
            pltpu.VMEM((per_w * 16,), jnp.float32),
            pltpu.VMEM((ROWS, D), jnp.float32),
            pltpu.VMEM((ROWS, D), jnp.float32),
            pltpu.VMEM((ROWS, D), jnp.float32),
            pltpu.VMEM((ROWS, D), jnp.float32),
            pltpu.SemaphoreType.DMA,
            pltpu.SemaphoreType.DMA,
            pltpu.SemaphoreType.DMA,
            pltpu.SemaphoreType.DMA,
            pltpu.SemaphoreType.DMA,
            pltpu.SemaphoreType.DMA,
        ],
    )
    def k(tab_hbm, idx_hbm, sc_hbm, sh_hbm, out_hbm, idx_v, sc_v,
          rows0, rows1, sh0, sh1, g0, g1, s0, s1, st0, st1):
        wid = lax.axis_index("s") * info.num_cores + lax.axis_index("c")
        w_base = pl.multiple_of(wid * per_w, per_w)
        pltpu.sync_copy(idx_hbm.at[pl.ds(w_base, per_w)], idx_v)
        pltpu.sync_copy(sc_hbm.at[pl.ds(w_base * 16, per_w * 16)], sc_v)

        rows = (rows0, rows1)
        shb = (sh0, sh1)
        gsem = (g0, g1)
        ssem = (s0, s1)
        stsem = (st0, st1)

        def issue(c, b):
            off = c * ROWS
            gh = pltpu.async_copy(tab_hbm.at[idx_v.at[pl.ds(off, ROWS)]],
                                  rows[b], gsem[b])
            sh_h = pltpu.async_copy(sh_hbm.at[pl.ds(w_base + off, ROWS)],
                                    shb[b], ssem[b])
            return gh, sh_h

        def drain(b):
            pltpu.make_async_copy(tab_hbm.at[idx_v.at[pl.ds(0, ROWS)]],
                                  rows[b], gsem[b]).wait()
            pltpu.make_async_copy(sh_hbm.at[pl.ds(w_base, ROWS)], shb[b],
                                  ssem[b]).wait()

        def wait_store(b):
            pltpu.make_async_copy(rows[b],
                                  out_hbm.at[pl.ds(w_base, ROWS)],
                                  stsem[b]).wait()

        def combine(c, b):
            @plsc.parallel_loop(0, ROWS, 1, unroll=2)
            def rb(r):
                g = sc_v[pl.ds((c * ROWS + r) * 16, 16)]
                for d in range(D // 16):
                    sl = pl.ds(d * 16, 16)
                    rows[b][r, sl] = rows[b][r, sl] * g + shb[b][r, sl]

        def store(c, b):
            return pltpu.async_copy(rows[b],
                                    out_hbm.at[pl.ds(w_base + c * ROWS,
                                                     ROWS)], stsem[b])

        issue(0, 0)

        def body(i, carry):
            e = 2 * i

            @pl.when(i > 0)
            def _():
                wait_store(1)

            issue(e + 1, 1)
            drain(0)
            combine(e, 0)
            store(e, 0)
            drain(1)
            combine(e + 1, 1)
            store(e + 1, 1)
            wait_store(0)

            @pl.when(i + 1 < npairs)
            def _():
                issue(e + 2, 0)

            return carry

        lax.fori_loop(0, npairs, body, 0)
        wait_store(1)

    return k(table, idx, sc16.reshape(T * 16), shared)


def _ffn_body(disp_ref, w1_ref, b1_ref, w2_ref, b2_ref, out_ref):
    x = disp_ref[0]
    h = jnp.dot(x, w1_ref[0], preferred_element_type=jnp.float32) + b1_ref[0]
    h = h * jax.nn.sigmoid(h)
    out_ref[0] = (jnp.dot(h, w2_ref[0], preferred_element_type=jnp.float32)
                  + b2_ref[0])


def _run_ffn(disp, W1, b1, W2, b2, E, CAP, CAP2):
    D = disp.shape[-1]
    F = W1.shape[-1]
    return pl.pallas_call(
        _ffn_body,
        grid=(E,),
        in_specs=[
            pl.BlockSpec((1, CAP, D), lambda e: (e, 0, 0)),
            pl.BlockSpec((1, D, F), lambda e: (e, 0, 0)),
            pl.BlockSpec((1, 1, F), lambda e: (e, 0, 0)),
            pl.BlockSpec((1, F, D), lambda e: (e, 0, 0)),
            pl.BlockSpec((1, 1, D), lambda e: (e, 0, 0)),
        ],
        out_specs=pl.BlockSpec((1, CAP, D), lambda e: (e, 0, 0)),
        out_shape=jax.ShapeDtypeStruct((E, CAP, D), jnp.float32),
    )(disp.reshape(E, CAP2, D), W1, b1.reshape(E, 1, F), W2,
      b2.reshape(E, 1, D))


def kernel(hidden_states, Wr, W1, b1, W2, b2, Ws1, bs1, Ws2, bs2):
    Bz, Sz, D = hidden_states.shape
    T = Bz * Sz
    E = Wr.shape[1]
    CAP = 2 * (T // E)
    CAP2 = CAP + 8
    x = hidden_states.reshape(T, D)

    dd, dg, sc16, sh = _run_router(x, Wr, Ws1, bs1, Ws2, bs2, E, CAP, CAP2,
                                   C=1024)
    disp = _sc_scatter_rows(x, dd, E * CAP2)
    eout = _run_ffn(disp, W1, b1, W2, b2, E, CAP, CAP2)
    out = _sc_gather_combine(eout.reshape(E * CAP, D), dg, sc16, sh)
    return out.reshape(Bz, Sz, D)


# speedup vs baseline: 1.1215x; 1.1215x over previous
import functools

import jax
import jax.numpy as jnp
from jax import lax
from jax.experimental import pallas as pl
from jax.experimental.pallas import tpu as pltpu
from jax.experimental.pallas import tpu_sc as plsc

SHARED_SCALE = 0.1


def _router_body(E, CAP, CAP2, C, x_ref, wr_ref, dd_ref, dg_ref, sc_ref,
                 counts_ref, tril_ref):
    step = pl.program_id(0)

    @pl.when(step == 0)
    def _init():
        counts_ref[...] = jnp.zeros_like(counts_ref)
        r = lax.broadcasted_iota(jnp.int32, (C, C), 0)
        c = lax.broadcasted_iota(jnp.int32, (C, C), 1)
        tril_ref[...] = (r >= c).astype(jnp.float32)

    x = x_ref[...]
    logits = jnp.dot(x, wr_ref[...],
                     preferred_element_type=jnp.float32)
    m = jnp.max(logits, axis=1, keepdims=True)
    iota_e = lax.broadcasted_iota(jnp.int32, logits.shape, 1)
    eidx = jnp.min(jnp.where(logits == m, iota_e, E), axis=1,
                   keepdims=True)
    gate = 1.0 / jnp.sum(jnp.exp(logits - m), axis=1, keepdims=True)

    oh = (iota_e == eidx).astype(jnp.float32)
    cum = jnp.dot(tril_ref[...], oh,
                  preferred_element_type=jnp.float32)
    pos = jnp.sum(cum * oh, axis=1, keepdims=True)
    base = jnp.sum(counts_ref[...] * oh, axis=1, keepdims=True)
    counts_ref[...] += jnp.sum(oh, axis=0, keepdims=True)

    sloti = (base + pos - 1.0).astype(jnp.int32)
    keep = sloti < CAP
    slot_c = jnp.minimum(sloti, CAP - 1)
    dd_ref[...] = eidx * CAP2 + jnp.minimum(sloti, CAP)
    dg_ref[...] = eidx * CAP + slot_c
    sc_ref[...] = jnp.where(keep, gate, 0.0)


def _run_router(x, Wr, E, CAP, CAP2, C):
    T, D = x.shape
    body = functools.partial(_router_body, E, CAP, CAP2, C)
    out_shape = [
        jax.ShapeDtypeStruct((T, 1), jnp.int32),
        jax.ShapeDtypeStruct((T, 1), jnp.int32),
        jax.ShapeDtypeStruct((T, 1), jnp.float32),
    ]
    col_spec = pl.BlockSpec((C, 1), lambda i: (i, 0))
    dd, dg, sc = pl.pallas_call(
        body,
        grid=(T // C,),
        in_specs=[
            pl.BlockSpec((C, D), lambda i: (i, 0)),
            pl.BlockSpec((D, E), lambda i: (0, 0)),
        ],
        out_specs=[col_spec, col_spec, col_spec],
        out_shape=out_shape,
        scratch_shapes=[pltpu.VMEM((1, E), jnp.float32),
                        pltpu.VMEM((C, C), jnp.float32)],
    )(x, Wr)
    return dd.reshape(T), dg.reshape(T), sc


def _sc_scatter_rows(x, idx, n_rows_out):
    T, D = x.shape
    info = plsc.get_sparse_core_info()
    NW = info.num_cores * info.num_subcores
    ROWS = 32
    per_w = T // NW
    nchunks = per_w // ROWS
    mesh = plsc.VectorSubcoreMesh(core_axis_name="c", subcore_axis_name="s")

    @functools.partial(
        pl.kernel,
        mesh=mesh,
        out_type=jax.ShapeDtypeStruct((n_rows_out, D), x.dtype),
        scratch_types=[
            pltpu.VMEM((ROWS,), jnp.int32),
            pltpu.VMEM((ROWS,), jnp.int32),
            pltpu.VMEM((ROWS, D), x.dtype),
            pltpu.VMEM((ROWS, D), x.dtype),
            pltpu.SemaphoreType.DMA,
            pltpu.SemaphoreType.DMA,
            pltpu.SemaphoreType.DMA,
            pltpu.SemaphoreType.DMA,
        ],
    )
    def k(x_hbm, idx_hbm, out_hbm, idx_v0, idx_v1, rows_v0, rows_v1,
          lsem0, lsem1, ssem0, ssem1):
        wid = lax.axis_index("s") * info.num_cores + lax.axis_index("c")
        w_base = wid * per_w
        idx_v = (idx_v0, idx_v1)
        rows_v = (rows_v0, rows_v1)
        lsem = (lsem0, lsem1)
        ssem = (ssem0, ssem1)

        def load(i):
            base = pl.multiple_of(w_base + i * ROWS, ROWS)
            b = i % 2
            hi = pltpu.async_copy(idx_hbm.at[pl.ds(base, ROWS)],
                                  idx_v[b], lsem[b])
            hr = pltpu.async_copy(x_hbm.at[pl.ds(base, ROWS)], rows_v[b],
                                  lsem[b])
            return hi, hr

        handles = load(0)
        scat = [None, None]
        for i in range(nchunks):
            b = i % 2
            handles[0].wait()
            handles[1].wait()
            scat[b] = pltpu.async_copy(rows_v[b], out_hbm.at[idx_v[b]],
                                       ssem[b])
            if i + 1 < nchunks:
                if scat[1 - b] is not None:
                    scat[1 - b].wait()
                handles = load(i + 1)
        if nchunks > 1:
            scat[(nchunks - 2) % 2].wait()
        scat[(nchunks - 1) % 2].wait()

    return k(x, idx)


def _sc_gather_rows(table, idx):
    _, D = table.shape
    T = idx.shape[0]
    info = plsc.get_sparse_core_info()
    NW = info.num_cores * info.num_subcores
    ROWS = 32
    per_w = T // NW
    nchunks = per_w // ROWS
    mesh = plsc.VectorSubcoreMesh(core_axis_name="c", subcore_axis_name="s")

    @functools.partial(
        pl.kernel,
        mesh=mesh,
        out_type=jax.ShapeDtypeStruct((T, D), table.dtype),
        scratch_types=[
            pltpu.VMEM((ROWS,), jnp.int32),
            pltpu.VMEM((ROWS,), jnp.int32),
            pltpu.VMEM((ROWS, D), table.dtype),
            pltpu.VMEM((ROWS, D), table.dtype),
            pltpu.SemaphoreType.DMA,
            pltpu.SemaphoreType.DMA,
            pltpu.SemaphoreType.DMA,
            pltpu.SemaphoreType.DMA,
        ],
    )
    def k(tab_hbm, idx_hbm, out_hbm, idx_v0, idx_v1, rows_v0, rows_v1,
          isem0, isem1, gsem0, gsem1):
        wid = lax.axis_index("s") * info.num_cores + lax.axis_index("c")
        w_base = wid * per_w
        idx_v = (idx_v0, idx_v1)
        rows_v = (rows_v0, rows_v1)
        isem = (isem0, isem1)
        gsem = (gsem0, gsem1)

        def start_gather(i):
            base = pl.multiple_of(w_base + i * ROWS, ROWS)
            b = i % 2
            pltpu.async_copy(idx_hbm.at[pl.ds(base, ROWS)], idx_v[b],
                             isem[b]).wait()
            return pltpu.async_copy(tab_hbm.at[idx_v[b]], rows_v[b],
                                    gsem[b])

        gh = start_gather(0)
        store = [None, None]
        for i in range(nchunks):
            b = i % 2
            gh.wait()
            if i + 1 < nchunks:
                if store[1 - b] is not None:
                    store[1 - b].wait()
                gh_next = start_gather(i + 1)
            base = pl.multiple_of(w_base + i * ROWS, ROWS)
            store[b] = pltpu.async_copy(rows_v[b],
                                        out_hbm.at[pl.ds(base, ROWS)],
                                        isem[b])
            if i + 1 < nchunks:
                gh = gh_next
        store[(nchunks - 1) % 2].wait()
        if nchunks > 1:
            store[(nchunks - 2) % 2].wait()

    return k(table, idx)


def _ffn_body(disp_ref, w1_ref, b1_ref, w2_ref, b2_ref, out_ref):
    x = disp_ref[0]
    h = jnp.dot(x, w1_ref[0], preferred_element_type=jnp.float32) + b1_ref[0]
    h = h * jax.nn.sigmoid(h)
    out_ref[0] = (jnp.dot(h, w2_ref[0], preferred_element_type=jnp.float32)
                  + b2_ref[0])


def _run_ffn(disp, W1, b1, W2, b2, E, CAP, CAP2):
    D = disp.shape[-1]
    F = W1.shape[-1]
    return pl.pallas_call(
        _ffn_body,
        grid=(E,),
        in_specs=[
            pl.BlockSpec((1, CAP, D), lambda e: (e, 0, 0)),
            pl.BlockSpec((1, D, F), lambda e: (e, 0, 0)),
            pl.BlockSpec((1, 1, F), lambda e: (e, 0, 0)),
            pl.BlockSpec((1, F, D), lambda e: (e, 0, 0)),
            pl.BlockSpec((1, 1, D), lambda e: (e, 0, 0)),
        ],
        out_specs=pl.BlockSpec((1, CAP, D), lambda e: (e, 0, 0)),
        out_shape=jax.ShapeDtypeStruct((E, CAP, D), jnp.float32),
    )(disp.reshape(E, CAP2, D), W1, b1.reshape(E, 1, F), W2,
      b2.reshape(E, 1, D))


def _shared_body(x_ref, moe_ref, sc_ref, ws1_ref, bs1_ref, ws2_ref, bs2_ref,
                 out_ref):
    x = x_ref[...]
    h = jnp.dot(x, ws1_ref[...], preferred_element_type=jnp.float32) + bs1_ref[0]
    h = h * jax.nn.sigmoid(h)
    shared = (jnp.dot(h, ws2_ref[...], preferred_element_type=jnp.float32)
              + bs2_ref[0])
    out_ref[...] = moe_ref[...] * sc_ref[...] + SHARED_SCALE * shared


def _run_shared_combine(x, moe, scale, Ws1, bs1, Ws2, bs2, C):
    T, D = x.shape
    F = Ws1.shape[-1]
    return pl.pallas_call(
        _shared_body,
        grid=(T // C,),
        in_specs=[
            pl.BlockSpec((C, D), lambda i: (i, 0)),
            pl.BlockSpec((C, D), lambda i: (i, 0)),
            pl.BlockSpec((C, 1), lambda i: (i, 0)),
            pl.BlockSpec((D, F), lambda i: (0, 0)),
            pl.BlockSpec((1, 1, F), lambda i: (0, 0, 0)),
            pl.BlockSpec((F, D), lambda i: (0, 0)),
            pl.BlockSpec((1, 1, D), lambda i: (0, 0, 0)),
        ],
        out_specs=pl.BlockSpec((C, D), lambda i: (i, 0)),
        out_shape=jax.ShapeDtypeStruct((T, D), jnp.float32),
    )(x, moe, scale, Ws1, bs1.reshape(1, 1, F), Ws2, bs2.reshape(1, 1, D))


def kernel(hidden_states, Wr, W1, b1, W2, b2, Ws1, bs1, Ws2, bs2):
    Bz, Sz, D = hidden_states.shape
    T = Bz * Sz
    E = Wr.shape[1]
    CAP = 2 * (T // E)
    CAP2 = CAP + 8
    x = hidden_states.reshape(T, D)

    dd, dg, scale = _run_router(x, Wr, E, CAP, CAP2, C=1024)
    disp = _sc_scatter_rows(x, dd, E * CAP2)
    eout = _run_ffn(disp, W1, b1, W2, b2, E, CAP, CAP2)
    moe = _sc_gather_rows(eout.reshape(E * CAP, D), dg)
    out = _run_shared_combine(x, moe, scale, Ws1, bs1, Ws2, bs2, C=1024)
    return out.reshape(Bz, Sz, D)
